# baseline (speedup 1.0000x reference)
import functools
import math

import jax
import jax.numpy as jnp
from jax import lax
from jax.experimental import pallas as pl
from jax.experimental.pallas import tpu as pltpu
from jax.experimental.pallas import tpu_sc as plsc

_EMB = 64
_SCALE = math.sqrt(_EMB)
_BSTEP = 256


@functools.lru_cache(maxsize=None)
def _build(batch, seq):
    info = plsc.get_sparse_core_info()
    nl = info.num_lanes
    nw = info.num_cores * info.num_subcores
    bpw = batch // nw
    hsteps = bpw // _BSTEP
    steps = seq * hsteps
    assert batch % (nw * _BSTEP) == 0 and steps % 2 == 0
    mesh = plsc.VectorSubcoreMesh(core_axis_name="c", subcore_axis_name="s")

    @functools.partial(
        pl.kernel,
        mesh=mesh,
        compiler_params=pltpu.CompilerParams(needs_layout_passes=False),
        out_type=jax.ShapeDtypeStruct((seq, _EMB, batch), jnp.float32),
        scratch_types=[
            pltpu.VMEM((2, 1, _BSTEP), jnp.int32),
            pltpu.VMEM((2, 1, _BSTEP), jnp.int32),
            pltpu.VMEM((2, _BSTEP, 2 * _EMB), jnp.float32),
            pltpu.VMEM((2, 1, _EMB, _BSTEP), jnp.float32),
            [pltpu.SemaphoreType.DMA] * 2,
            [pltpu.SemaphoreType.DMA] * 2,
            [pltpu.SemaphoreType.DMA] * 2,
        ],
    )
    def embed(idx_hbm, wide_hbm, out_hbm, idx_v, u_v, rows_v, outs_v,
              gsem, ssem, isem):
        wid = lax.axis_index("s") * info.num_cores + lax.axis_index("c")
        b0 = wid * bpw

        def coords(chunk):
            if hsteps == 1:
                return chunk, b0
            return chunk // hsteps, b0 + (chunk % hsteps) * _BSTEP

        def fire_idx(chunk, p):
            s, bb = coords(chunk)
            pltpu.make_async_copy(
                idx_hbm.at[pl.ds(s, 1), pl.ds(bb, _BSTEP)], idx_v.at[p], isem[p]
            ).start()

        def compute_pair_ids(p):
            for o in range(0, _BSTEP, nl):
                sl = pl.ds(o, nl)
                u_v[p, 0, sl] = lax.shift_right_logical(idx_v[p, 0, sl], 1)

        def fire_gathers(p):
            for g in range(_BSTEP // 128):
                pltpu.make_async_copy(
                    wide_hbm.at[u_v.at[p, 0, pl.ds(g * 128, 128)]],
                    rows_v.at[p, pl.ds(g * 128, 128)],
                    gsem[p],
                ).start()

        def drain_gathers(p):
            for g in range(_BSTEP // 128):
                pltpu.make_async_copy(
                    wide_hbm.at[u_v.at[p, 0, pl.ds(g * 128, 128)]],
                    rows_v.at[p, pl.ds(g * 128, 128)],
                    gsem[p],
                ).wait()

        def scatter(chunk, p, wait):
            s, bb = coords(chunk)
            cp = pltpu.make_async_copy(
                outs_v.at[p],
                out_hbm.at[pl.ds(s, 1), pl.ds(0, _EMB), pl.ds(bb, _BSTEP)],
                ssem[p],
            )
            cp.wait() if wait else cp.start()

        lanes = lax.iota(jnp.int32, nl)
        perms = [(lanes + d) & (nl - 1) for d in range(nl)]

        def step(chunk, p):
            q = 1 - p
            drain_gathers(p)
            @pl.when(jnp.logical_and(chunk > 0, chunk + 1 < steps))
            def _():
                scatter(chunk - 1, q, wait=True)

            @pl.when(chunk + 1 < steps)
            def _():
                s, bb = coords(chunk + 1)
                pltpu.make_async_copy(
                    idx_hbm.at[pl.ds(s, 1), pl.ds(bb, _BSTEP)],
                    idx_v.at[q], isem[q],
                ).wait()
                compute_pair_ids(q)
                fire_gathers(q)

            @pl.when(chunk + 2 < steps)
            def _():
                fire_idx(chunk + 2, p)

            outs2d = outs_v.at[p, 0]

            @plsc.parallel_loop(0, _BSTEP // nl, unroll=4)
            def _(c):
                rows16 = rows_v.at[p, pl.ds(c * nl, nl)]
                csl = c * nl + lanes
                half = (idx_v[p, 0, pl.ds(c * nl, nl)] & 1) * _EMB
                for e0 in range(0, _EMB, nl):
                    for d in range(nl):
                        cols = half + (e0 + perms[d])
                        vals = plsc.load_gather(rows16, [lanes, cols])
                        plsc.store_scatter(
                            outs2d, [e0 + perms[d], csl], vals * _SCALE
                        )

            scatter(chunk, p, wait=False)

        s0, bb0 = coords(0)
        pltpu.sync_copy(
            idx_hbm.at[pl.ds(s0, 1), pl.ds(bb0, _BSTEP)], idx_v.at[0]
        )
        compute_pair_ids(0)
        fire_gathers(0)
        fire_idx(1, 1)

        def pair(h, carry):
            step(2 * h, 0)
            step(2 * h + 1, 1)
            return carry

        lax.fori_loop(0, steps // 2, pair, 0)
        scatter(steps - 2, 0, wait=True)
        scatter(steps - 1, 1, wait=True)

    return embed


@jax.jit
def kernel(tokens, embedding_weight):
    batch, seq = tokens.shape
    vocab, emb = embedding_weight.shape
    tokens_t = jnp.transpose(tokens).astype(jnp.int32)
    wide = embedding_weight.reshape(vocab // 2, 2 * emb)
    out_t = _build(batch, seq)(tokens_t, wide)
    return jnp.transpose(out_t, (2, 0, 1))


# speedup vs baseline: 1.0467x; 1.0467x over previous
import functools
import math

import jax
import jax.numpy as jnp
from jax import lax
from jax.experimental import pallas as pl
from jax.experimental.pallas import tpu as pltpu
from jax.experimental.pallas import tpu_sc as plsc

_EMB = 64
_SCALE = math.sqrt(_EMB)
_BSTEP = 256


@functools.lru_cache(maxsize=None)
def _build(batch, seq):
    info = plsc.get_sparse_core_info()
    nl = info.num_lanes
    nw = info.num_cores * info.num_subcores
    bpw = batch // nw
    hsteps = bpw // _BSTEP
    steps = seq * hsteps
    assert batch % (nw * _BSTEP) == 0 and steps % 2 == 0
    mesh = plsc.VectorSubcoreMesh(core_axis_name="c", subcore_axis_name="s")

    @functools.partial(
        pl.kernel,
        mesh=mesh,
        compiler_params=pltpu.CompilerParams(needs_layout_passes=False),
        out_type=jax.ShapeDtypeStruct((seq, _EMB, batch), jnp.float32),
        scratch_types=[
            pltpu.VMEM((2, 1, _BSTEP), jnp.int32),
            pltpu.VMEM((2, _BSTEP, 2 * _EMB), jnp.float32),
            pltpu.VMEM((2, 1, _EMB, _BSTEP), jnp.float32),
            [pltpu.SemaphoreType.DMA] * 2,
            [pltpu.SemaphoreType.DMA] * 2,
            [pltpu.SemaphoreType.DMA] * 2,
        ],
    )
    def embed(idx_hbm, wide_hbm, out_hbm, idx_v, rows_v, outs_v,
              gsem, ssem, isem):
        wid = lax.axis_index("s") * info.num_cores + lax.axis_index("c")
        b0 = wid * bpw

        def coords(chunk):
            if hsteps == 1:
                return chunk, b0
            return chunk // hsteps, b0 + (chunk % hsteps) * _BSTEP

        def fire_idx(chunk, p):
            s, bb = coords(chunk)
            pltpu.make_async_copy(
                idx_hbm.at[pl.ds(s, 1), pl.ds(bb, _BSTEP)], idx_v.at[p], isem[p]
            ).start()

        def fire_gathers(p):
            for g in range(_BSTEP // 128):
                pltpu.make_async_copy(
                    wide_hbm.at[idx_v.at[p, 0, pl.ds(g * 128, 128)]],
                    rows_v.at[p, pl.ds(g * 128, 128)],
                    gsem[p],
                ).start()

        def drain_gathers(p):
            for g in range(_BSTEP // 128):
                pltpu.make_async_copy(
                    wide_hbm.at[idx_v.at[p, 0, pl.ds(g * 128, 128)]],
                    rows_v.at[p, pl.ds(g * 128, 128)],
                    gsem[p],
                ).wait()

        def scatter(chunk, p, wait):
            s, bb = coords(chunk)
            cp = pltpu.make_async_copy(
                outs_v.at[p],
                out_hbm.at[pl.ds(s, 1), pl.ds(0, _EMB), pl.ds(bb, _BSTEP)],
                ssem[p],
            )
            cp.wait() if wait else cp.start()

        lanes = lax.iota(jnp.int32, nl)
        perms = [(lanes + d) & (nl - 1) for d in range(nl)]

        def step(chunk, p):
            q = 1 - p
            drain_gathers(p)
            @pl.when(jnp.logical_and(chunk > 0, chunk + 1 < steps))
            def _():
                scatter(chunk - 1, q, wait=True)

            @pl.when(chunk + 1 < steps)
            def _():
                s, bb = coords(chunk + 1)
                pltpu.make_async_copy(
                    idx_hbm.at[pl.ds(s, 1), pl.ds(bb, _BSTEP)],
                    idx_v.at[q], isem[q],
                ).wait()
                fire_gathers(q)

            @pl.when(chunk + 2 < steps)
            def _():
                fire_idx(chunk + 2, p)

            outs2d = outs_v.at[p, 0]

            @plsc.parallel_loop(0, _BSTEP // nl, unroll=4)
            def _(c):
                rows16 = rows_v.at[p, pl.ds(c * nl, nl)]
                csl = c * nl + lanes
                for e0 in range(0, _EMB, nl):
                    for d in range(nl):
                        cols = e0 + perms[d]
                        vals = plsc.load_gather(rows16, [lanes, cols])
                        plsc.store_scatter(outs2d, [cols, csl], vals * _SCALE)

            scatter(chunk, p, wait=False)

        s0, bb0 = coords(0)
        pltpu.sync_copy(
            idx_hbm.at[pl.ds(s0, 1), pl.ds(bb0, _BSTEP)], idx_v.at[0]
        )
        fire_gathers(0)
        fire_idx(1, 1)

        def pair(h, carry):
            step(2 * h, 0)
            step(2 * h + 1, 1)
            return carry

        lax.fori_loop(0, steps // 2, pair, 0)
        scatter(steps - 2, 0, wait=True)
        scatter(steps - 1, 1, wait=True)

    return embed


@jax.jit
def kernel(tokens, embedding_weight):
    batch, seq = tokens.shape
    vocab, emb = embedding_weight.shape
    tokens_t = jnp.transpose(tokens).astype(jnp.int32)
    wide = jnp.pad(embedding_weight, ((0, 0), (0, emb)))
    out_t = _build(batch, seq)(tokens_t, wide)
    return jnp.transpose(out_t, (2, 0, 1))
